# Initial kernel scaffold; baseline (speedup 1.0000x reference)
import jax
import jax.numpy as jnp
from jax.experimental import pallas as pl


def kernel(t, vol0, vol1, vol2, vol3):
    raise NotImplementedError("write your pallas kernel here")


# speedup vs baseline: 3.8405x; 3.8405x over previous
import jax
import jax.numpy as jnp
from jax import lax
from jax.experimental import pallas as pl
from jax.experimental.pallas import tpu as pltpu
from jax.experimental.pallas import tpu_sc as plsc

SIZES = (128, 512, 2048, 8192)
C = 32
N = 262144

NC, NS, LANES = 2, 16, 16
NW = NC * NS
PPW = N // NW
CHUNK = 128
GROUPS = CHUNK // LANES
NCHUNK = PPW // CHUNK

RES_BASES = (0, 128, 640)
RES_ROWS = 128 + 512 + 2048


def _sc_body(t_hbm, tab_hbm, big3_hbm, out_hbm,
             tab_v, t_v, w_v, r_v, idx3, rows3, out_v, sem):
    wid = lax.axis_index("s") * NC + lax.axis_index("c")
    base = wid * PPW

    pltpu.sync_copy(tab_hbm, tab_v)

    def chunk_body(ci, _):
        cbase = base + ci * CHUNK
        pltpu.sync_copy(t_hbm.at[pl.ds(cbase, CHUNK)], t_v)

        def grp_idx(g, _):
            s = g * LANES
            tv = t_v[pl.ds(s, LANES)]
            tcl = jnp.minimum(jnp.maximum(tv, 0.0), 1.0)
            tc2 = tcl * 2.0 - 1.0
            for l in range(4):
                size = SIZES[l]
                pix = (tc2 + 1.0) * 0.5 * (size - 1)
                i0 = jnp.minimum(pix.astype(jnp.int32), size - 2)
                w = pix - i0.astype(jnp.float32)
                w_v[pl.ds(l * CHUNK + s, LANES)] = w
                if l < 3:
                    r_v[pl.ds(l * CHUNK + s, LANES)] = (i0 + RES_BASES[l]) * C
                else:
                    idx3[pl.ds(s, LANES)] = i0
            return 0

        lax.fori_loop(0, GROUPS, grp_idx, 0)

        pltpu.async_copy(big3_hbm.at[idx3], rows3, sem).wait()

        def grp_lerp(g, _):
            s = g * LANES
            pt = s + lax.iota(jnp.int32, LANES)
            obase = pt * (4 * C)
            for l in range(4):
                wv = w_v[pl.ds(l * CHUNK + s, LANES)]
                wc = 1.0 - wv
                if l < 3:
                    ra = r_v[pl.ds(l * CHUNK + s, LANES)]
                    rb = ra + C
                    for c in range(C):
                        a = plsc.load_gather(tab_v, [ra + c])
                        b = plsc.load_gather(tab_v, [rb + c])
                        o = a * wc + b * wv
                        plsc.store_scatter(out_v, [obase + (l * C + c)], o)
                else:
                    for c in range(C):
                        cv = jnp.full((LANES,), c, jnp.int32)
                        a = plsc.load_gather(rows3, [pt, cv])
                        b = plsc.load_gather(rows3, [pt, cv + C])
                        o = a * wc + b * wv
                        plsc.store_scatter(out_v, [obase + (l * C + c)], o)
            return 0

        lax.fori_loop(0, GROUPS, grp_lerp, 0)

        pltpu.sync_copy(out_v, out_hbm.at[pl.ds(cbase * (4 * C), CHUNK * 4 * C)])
        return 0

    lax.fori_loop(0, NCHUNK, chunk_body, 0)


@jax.jit
def _run(t_flat, tab012, big3):
    mesh = plsc.VectorSubcoreMesh(core_axis_name="c", subcore_axis_name="s",
                                  num_cores=NC, num_subcores=NS)
    f = pl.kernel(
        _sc_body,
        out_type=jax.ShapeDtypeStruct((N * 4 * C,), jnp.float32),
        mesh=mesh,
        scratch_types=[
            pltpu.VMEM((RES_ROWS * C,), jnp.float32),
            pltpu.VMEM((CHUNK,), jnp.float32),
            pltpu.VMEM((4 * CHUNK,), jnp.float32),
            pltpu.VMEM((3 * CHUNK,), jnp.int32),
            pltpu.VMEM((CHUNK,), jnp.int32),
            pltpu.VMEM((CHUNK, 2 * C), jnp.float32),
            pltpu.VMEM((CHUNK * 4 * C,), jnp.float32),
            pltpu.SemaphoreType.DMA,
        ],
        compiler_params=pltpu.CompilerParams(needs_layout_passes=False,
                                             use_tc_tiling_on_sc=False),
    )
    return f(t_flat, tab012, big3)


def kernel(t, vol0, vol1, vol2, vol3):
    t_flat = t[:, 0]
    v0 = vol0[0, :, :, 0].T
    v1 = vol1[0, :, :, 0].T
    v2 = vol2[0, :, :, 0].T
    v3 = vol3[0, :, :, 0].T
    tab012 = jnp.concatenate([v0, v1, v2], axis=0).reshape(-1)
    big3 = jnp.concatenate([v3[:-1], v3[1:]], axis=1)
    out = _run(t_flat, tab012, big3)
    return out.reshape(N, 4 * C)


# speedup vs baseline: 5.6197x; 1.4633x over previous
import jax
import jax.numpy as jnp
from jax import lax
from jax.experimental import pallas as pl
from jax.experimental.pallas import tpu as pltpu
from jax.experimental.pallas import tpu_sc as plsc

SIZES = (128, 512, 2048, 8192)
C = 32
N = 262144

NC, NS, LANES = 2, 16, 16
NW = NC * NS
PPW = N // NW
CHUNK = 64
GROUPS = CHUNK // LANES
NCHUNK = PPW // CHUNK

RES_BASES = (0, 128, 640)
RES_ROWS = 128 + 512 + 2048


def _sc_body(t_hbm, tab_hbm, big3_hbm, out_hbm,
             tab_v, t_v, w_v0, w_v1, r_v0, r_v1, i3_0, i3_1,
             rows3_0, rows3_1, out_v0, out_v1, gsem0, gsem1, osem0, osem1):
    wid = lax.axis_index("s") * NC + lax.axis_index("c")
    base = wid * PPW

    pltpu.sync_copy(tab_hbm, tab_v)
    pltpu.sync_copy(t_hbm.at[pl.ds(base, PPW)], t_v)

    w_v = (w_v0, w_v1)
    r_v = (r_v0, r_v1)
    i3 = (i3_0, i3_1)
    rows3 = (rows3_0, rows3_1)
    out_v = (out_v0, out_v1)
    gsem = (gsem0, gsem1)
    osem = (osem0, osem1)

    def compute_idx(ci, nb):
        coff = ci * CHUNK

        @plsc.parallel_loop(0, GROUPS)
        def _(g):
            s = g * LANES
            tv = t_v[pl.ds(coff + s, LANES)]
            tcl = jnp.minimum(jnp.maximum(tv, 0.0), 1.0)
            tc2 = tcl * 2.0 - 1.0
            for l in range(4):
                size = SIZES[l]
                pix = (tc2 + 1.0) * 0.5 * (size - 1)
                i0 = jnp.minimum(pix.astype(jnp.int32), size - 2)
                w = pix - i0.astype(jnp.float32)
                w_v[nb][pl.ds(l * CHUNK + s, LANES)] = w
                if l < 3:
                    r_v[nb][pl.ds(l * CHUNK + s, LANES)] = (i0 + RES_BASES[l]) * C
                else:
                    i3[nb][pl.ds(s, LANES)] = i0

    def issue_gather(nb):
        pltpu.async_copy(big3_hbm.at[i3[nb]], rows3[nb], gsem[nb])

    def wait_gather(b):
        pltpu.make_async_copy(big3_hbm.at[i3[b]], rows3[b], gsem[b]).wait()

    KB = 8

    def lerp(b):
        @plsc.parallel_loop(0, GROUPS)
        def _(g):
            s = g * LANES
            pt = s + lax.iota(jnp.int32, LANES)
            obase = pt * (4 * C)
            for l in range(4):
                wv = w_v[b][pl.ds(l * CHUNK + s, LANES)]
                wc = 1.0 - wv
                if l < 3:
                    ra = r_v[b][pl.ds(l * CHUNK + s, LANES)]
                    rb = ra + C

                    def ld(c, ra=ra, rb=rb):
                        return (plsc.load_gather(tab_v, [ra + c]),
                                plsc.load_gather(tab_v, [rb + c]))
                else:

                    def ld(c, pt=pt):
                        cv = jnp.full((LANES,), c, jnp.int32)
                        return (plsc.load_gather(rows3[b], [pt, cv]),
                                plsc.load_gather(rows3[b], [pt, cv + C]))

                cur = [ld(c) for c in range(KB)]
                for c0 in range(0, C, KB):
                    nxt = ([ld(c) for c in range(c0 + KB, c0 + 2 * KB)]
                           if c0 + KB < C else [])
                    for i, c in enumerate(range(c0, c0 + KB)):
                        a, bb = cur[i]
                        o = a * wc + bb * wv
                        plsc.store_scatter(out_v[b], [obase + (l * C + c)], o)
                    cur = nxt

    def issue_store(ci, b):
        cbase = (base + ci * CHUNK) * (4 * C)
        pltpu.async_copy(out_v[b], out_hbm.at[pl.ds(cbase, CHUNK * 4 * C)], osem[b])

    def wait_store(ci, b):
        cbase = (base + ci * CHUNK) * (4 * C)
        pltpu.make_async_copy(out_v[b], out_hbm.at[pl.ds(cbase, CHUNK * 4 * C)],
                              osem[b]).wait()

    def substep(ci, b, nb, do_prefetch, do_wait_store):
        if do_prefetch:
            compute_idx(ci + 1, nb)
            issue_gather(nb)
        wait_gather(b)
        if do_wait_store is None:
            wait_store(ci, b)
        elif do_wait_store is not False:
            @pl.when(do_wait_store)
            def _():
                wait_store(ci, b)
        lerp(b)
        issue_store(ci, b)

    compute_idx(0, 0)
    issue_gather(0)

    def pair_body(i, _):
        ci = 2 * i
        substep(ci, 0, 1, True, i >= 1)
        substep(ci + 1, 1, 0, True, i >= 1)
        return 0

    lax.fori_loop(0, NCHUNK // 2 - 1, pair_body, 0)

    substep(NCHUNK - 2, 0, 1, True, None)
    substep(NCHUNK - 1, 1, 0, False, None)
    wait_store(NCHUNK - 2, 0)
    wait_store(NCHUNK - 1, 1)


@jax.jit
def _run(t_flat, tab012, big3):
    mesh = plsc.VectorSubcoreMesh(core_axis_name="c", subcore_axis_name="s",
                                  num_cores=NC, num_subcores=NS)
    f = pl.kernel(
        _sc_body,
        out_type=jax.ShapeDtypeStruct((N * 4 * C,), jnp.float32),
        mesh=mesh,
        scratch_types=[
            pltpu.VMEM((RES_ROWS * C,), jnp.float32),
            pltpu.VMEM((PPW,), jnp.float32),
            pltpu.VMEM((4 * CHUNK,), jnp.float32),
            pltpu.VMEM((4 * CHUNK,), jnp.float32),
            pltpu.VMEM((3 * CHUNK,), jnp.int32),
            pltpu.VMEM((3 * CHUNK,), jnp.int32),
            pltpu.VMEM((CHUNK,), jnp.int32),
            pltpu.VMEM((CHUNK,), jnp.int32),
            pltpu.VMEM((CHUNK, 2 * C), jnp.float32),
            pltpu.VMEM((CHUNK, 2 * C), jnp.float32),
            pltpu.VMEM((CHUNK * 4 * C,), jnp.float32),
            pltpu.VMEM((CHUNK * 4 * C,), jnp.float32),
            pltpu.SemaphoreType.DMA,
            pltpu.SemaphoreType.DMA,
            pltpu.SemaphoreType.DMA,
            pltpu.SemaphoreType.DMA,
        ],
        compiler_params=pltpu.CompilerParams(needs_layout_passes=False,
                                             use_tc_tiling_on_sc=False),
    )
    return f(t_flat, tab012, big3)


def kernel(t, vol0, vol1, vol2, vol3):
    t_flat = t[:, 0]
    v0 = vol0[0, :, :, 0].T
    v1 = vol1[0, :, :, 0].T
    v2 = vol2[0, :, :, 0].T
    v3 = vol3[0, :, :, 0].T
    tab012 = jnp.concatenate([v0, v1, v2], axis=0).reshape(-1)
    big3 = jnp.concatenate([v3[:-1], v3[1:]], axis=1)
    out = _run(t_flat, tab012, big3)
    return out.reshape(N, 4 * C)


# speedup vs baseline: 11.1187x; 1.9785x over previous
import jax
import jax.numpy as jnp
from jax import lax
from jax.experimental import pallas as pl
from jax.experimental.pallas import tpu as pltpu
from jax.experimental.pallas import tpu_sc as plsc

SIZES = (128, 512, 2048, 8192)
C = 32
N = 262144

NC, NS, LANES = 2, 16, 16
NW = NC * NS
PPW = N // NW
CHUNK = 128
GROUPS = CHUNK // LANES
NCHUNK = PPW // CHUNK

RES_BASES = (0, 128, 640)
RES_ROWS = 128 + 512 + 2048
CW = C // 2


def _sc_body(t_hbm, tab_hbm, big3_hbm, out_hbm,
             tab_v, t_v, w_v0, w_v1, r_v0, r_v1, i3_0, i3_1,
             rows3_0, rows3_1, out_v0, out_v1, gsem0, gsem1, osem0, osem1):
    wid = lax.axis_index("s") * NC + lax.axis_index("c")
    base = wid * PPW

    pltpu.sync_copy(tab_hbm, tab_v)
    pltpu.sync_copy(t_hbm.at[pl.ds(base, PPW)], t_v)

    w_v = (w_v0, w_v1)
    r_v = (r_v0, r_v1)
    i3 = (i3_0, i3_1)
    rows3 = (rows3_0, rows3_1)
    out_v = (out_v0, out_v1)
    gsem = (gsem0, gsem1)
    osem = (osem0, osem1)

    def compute_idx(ci, nb):
        coff = ci * CHUNK

        @plsc.parallel_loop(0, GROUPS)
        def _(g):
            s = g * LANES
            tv = t_v[pl.ds(coff + s, LANES)]
            tcl = jnp.minimum(jnp.maximum(tv, 0.0), 1.0)
            tc2 = tcl * 2.0 - 1.0
            for l in range(4):
                size = SIZES[l]
                pix = (tc2 + 1.0) * 0.5 * (size - 1)
                i0 = jnp.minimum(pix.astype(jnp.int32), size - 2)
                w = pix - i0.astype(jnp.float32)
                w_v[nb][pl.ds(l * CHUNK + s, LANES)] = w
                if l < 3:
                    r_v[nb][pl.ds(l * CHUNK + s, LANES)] = (i0 + RES_BASES[l]) * CW
                else:
                    i3[nb][pl.ds(s, LANES)] = i0

    def issue_gather(nb):
        pltpu.async_copy(big3_hbm.at[i3[nb]], rows3[nb], gsem[nb])

    def wait_gather(b):
        pltpu.make_async_copy(big3_hbm.at[i3[b]], rows3[b], gsem[b]).wait()

    KB = 8

    def lerp(b):
        @plsc.parallel_loop(0, GROUPS)
        def _(g):
            s = g * LANES
            pt = s + lax.iota(jnp.int32, LANES)
            obase = pt * (4 * C)
            for l in range(4):
                wv = w_v[b][pl.ds(l * CHUNK + s, LANES)]
                wc = 1.0 - wv
                wv2 = plsc.pack(wv, wv, format=plsc.PackFormat.INTERLEAVED)
                wc2 = plsc.pack(wc, wc, format=plsc.PackFormat.INTERLEAVED)
                if l < 3:
                    ra = r_v[b][pl.ds(l * CHUNK + s, LANES)]
                    rb = ra + CW

                    def ld(c, ra=ra, rb=rb):
                        return (plsc.load_gather(tab_v, [ra + c]),
                                plsc.load_gather(tab_v, [rb + c]))
                else:

                    def ld(c, pt=pt):
                        cv = jnp.full((LANES,), c, jnp.int32)
                        return (plsc.load_gather(rows3[b], [pt, cv]),
                                plsc.load_gather(rows3[b], [pt, cv + CW]))

                cur = [ld(c) for c in range(KB)]
                for c0 in range(0, CW, KB):
                    nxt = ([ld(c) for c in range(c0 + KB, c0 + 2 * KB)]
                           if c0 + KB < CW else [])
                    for i, c in enumerate(range(c0, c0 + KB)):
                        aw, bw = cur[i]
                        af = plsc.bitcast(aw, jnp.bfloat16)
                        bf = plsc.bitcast(bw, jnp.bfloat16)
                        o2 = af * wc2 + bf * wv2
                        o0, o1 = plsc.unpack(o2, format=plsc.PackFormat.INTERLEAVED)
                        col = obase + (l * C + 2 * c)
                        plsc.store_scatter(out_v[b], [col], o0)
                        plsc.store_scatter(out_v[b], [col + 1], o1)
                    cur = nxt

    def issue_store(ci, b):
        cbase = (base + ci * CHUNK) * (4 * C)
        pltpu.async_copy(out_v[b], out_hbm.at[pl.ds(cbase, CHUNK * 4 * C)], osem[b])

    def wait_store(ci, b):
        cbase = (base + ci * CHUNK) * (4 * C)
        pltpu.make_async_copy(out_v[b], out_hbm.at[pl.ds(cbase, CHUNK * 4 * C)],
                              osem[b]).wait()

    def substep(ci, b, nb, do_prefetch, do_wait_store):
        if do_prefetch:
            compute_idx(ci + 1, nb)
            issue_gather(nb)
        wait_gather(b)
        if do_wait_store is None:
            wait_store(ci, b)
        elif do_wait_store is not False:
            @pl.when(do_wait_store)
            def _():
                wait_store(ci, b)
        lerp(b)
        issue_store(ci, b)

    compute_idx(0, 0)
    issue_gather(0)

    def pair_body(i, _):
        ci = 2 * i
        substep(ci, 0, 1, True, i >= 1)
        substep(ci + 1, 1, 0, True, i >= 1)
        return 0

    lax.fori_loop(0, NCHUNK // 2 - 1, pair_body, 0)

    substep(NCHUNK - 2, 0, 1, True, None)
    substep(NCHUNK - 1, 1, 0, False, None)
    wait_store(NCHUNK - 2, 0)
    wait_store(NCHUNK - 1, 1)


@jax.jit
def _run(t_flat, tab012, big3):
    mesh = plsc.VectorSubcoreMesh(core_axis_name="c", subcore_axis_name="s",
                                  num_cores=NC, num_subcores=NS)
    f = pl.kernel(
        _sc_body,
        out_type=jax.ShapeDtypeStruct((N * 4 * C,), jnp.float32),
        mesh=mesh,
        scratch_types=[
            pltpu.VMEM((RES_ROWS * CW,), jnp.int32),
            pltpu.VMEM((PPW,), jnp.float32),
            pltpu.VMEM((4 * CHUNK,), jnp.float32),
            pltpu.VMEM((4 * CHUNK,), jnp.float32),
            pltpu.VMEM((3 * CHUNK,), jnp.int32),
            pltpu.VMEM((3 * CHUNK,), jnp.int32),
            pltpu.VMEM((CHUNK,), jnp.int32),
            pltpu.VMEM((CHUNK,), jnp.int32),
            pltpu.VMEM((CHUNK, 2 * CW), jnp.int32),
            pltpu.VMEM((CHUNK, 2 * CW), jnp.int32),
            pltpu.VMEM((CHUNK * 4 * C,), jnp.float32),
            pltpu.VMEM((CHUNK * 4 * C,), jnp.float32),
            pltpu.SemaphoreType.DMA,
            pltpu.SemaphoreType.DMA,
            pltpu.SemaphoreType.DMA,
            pltpu.SemaphoreType.DMA,
        ],
        compiler_params=pltpu.CompilerParams(needs_layout_passes=False,
                                             use_tc_tiling_on_sc=False),
    )
    return f(t_flat, tab012, big3)


def _pack_rows(x):
    xb = x.astype(jnp.bfloat16)
    return jax.lax.bitcast_convert_type(
        xb.reshape(x.shape[0], -1, 2), jnp.int32)


def kernel(t, vol0, vol1, vol2, vol3):
    t_flat = t[:, 0]
    v0 = vol0[0, :, :, 0].T
    v1 = vol1[0, :, :, 0].T
    v2 = vol2[0, :, :, 0].T
    v3 = vol3[0, :, :, 0].T
    tab012 = _pack_rows(jnp.concatenate([v0, v1, v2], axis=0)).reshape(-1)
    v3p = _pack_rows(v3)
    big3 = jnp.concatenate([v3p[:-1], v3p[1:]], axis=1)
    out = _run(t_flat, tab012, big3)
    return out.reshape(N, 4 * C)
